# baseline (speedup 1.0000x reference)
import functools

import jax
import jax.numpy as jnp
from jax import lax
from jax.experimental import pallas as pl
from jax.experimental.pallas import tpu as pltpu
from jax.experimental.pallas import tpu_sc as plsc

D = 64
NC, NS = 2, 16
NW = NC * NS
CH = 128
NBUF = 4


def _sc_gather(table, idx3):
    nw, nch, ch = idx3.shape
    n = nw * nch * ch
    ring = 2 * NBUF
    ngrp = nch // ring
    mesh = plsc.VectorSubcoreMesh(core_axis_name="c", subcore_axis_name="s")

    @functools.partial(
        pl.kernel,
        mesh=mesh,
        compiler_params=pltpu.CompilerParams(use_tc_tiling_on_sc=False),
        out_type=jax.ShapeDtypeStruct((n, D), jnp.float32),
        scratch_types=[
            pltpu.VMEM((nch, ch), jnp.int32),
            pltpu.VMEM((ring, ch, D), jnp.float32),
        ] + [pltpu.SemaphoreType.DMA] * (2 * ring),
    )
    def k(table_hbm, idx_hbm, out_hbm, idx_v, rows_v, *sems):
        gsems, osems = sems[:ring], sems[ring:]
        wid = lax.axis_index("s") * NC + lax.axis_index("c")
        base = wid * (nch * ch)
        pltpu.sync_copy(idx_hbm.at[wid], idx_v)

        def gstart(j, b):
            pltpu.async_copy(table_hbm.at[idx_v.at[j]], rows_v.at[b], gsems[b])

        def gwait(j, b):
            pltpu.make_async_copy(
                table_hbm.at[idx_v.at[j]], rows_v.at[b], gsems[b]
            ).wait()

        def ostart(j, b):
            pltpu.async_copy(
                rows_v.at[b], out_hbm.at[pl.ds(base + j * ch, ch)], osems[b]
            )

        def owait(j, b):
            pltpu.make_async_copy(
                rows_v.at[b], out_hbm.at[pl.ds(base + j * ch, ch)], osems[b]
            ).wait()

        for b in range(NBUF):
            gstart(b, b)

        def body(g, carry):
            for b in range(ring):
                j = g * ring + b
                gwait(j, b)
                ostart(j, b)
                bn = (b + NBUF) % ring
                jn = j + NBUF
                if b < NBUF:
                    @pl.when(g >= 1)
                    def _():
                        owait(jn - ring, bn)
                        gstart(jn, bn)

                    @pl.when(g < 1)
                    def _():
                        gstart(jn, bn)
                else:

                    @pl.when(g < ngrp - 1)
                    def _():
                        owait(jn - ring, bn)
                        gstart(jn, bn)
            return carry

        lax.fori_loop(0, ngrp, body, 0)

        for b in range(ring):
            owait(nch - ring + b, b)

    return k(table, idx3)


def kernel(variate_ids, variate_embed_weight):
    b, h = variate_ids.shape
    n = b * h
    idx3 = variate_ids.reshape(NW, n // (NW * CH), CH).astype(jnp.int32)
    out = _sc_gather(variate_embed_weight, idx3)
    return out.reshape(b, h, D)


# speedup vs baseline: 1.7682x; 1.7682x over previous
import functools

import jax
import jax.numpy as jnp
from jax import lax
from jax.experimental import pallas as pl
from jax.experimental.pallas import tpu as pltpu
from jax.experimental.pallas import tpu_sc as plsc

D = 64
NC, NS = 2, 16
NW = NC * NS
CH = 128
NBUF = 4


def _sc_gather(table, idx3):
    nw, nch, ch = idx3.shape
    n = nw * nch * ch
    ring = 2 * NBUF
    ngrp = nch // ring
    mesh = plsc.VectorSubcoreMesh(core_axis_name="c", subcore_axis_name="s")

    @functools.partial(
        pl.kernel,
        mesh=mesh,
        compiler_params=pltpu.CompilerParams(use_tc_tiling_on_sc=False),
        out_type=jax.ShapeDtypeStruct((n, 128), jnp.float32),
        scratch_types=[
            pltpu.VMEM((nch, ch), jnp.int32),
            pltpu.VMEM((ring, ch, D), jnp.float32),
        ] + [pltpu.SemaphoreType.DMA] * (2 * ring),
    )
    def k(table_hbm, idx_hbm, out_hbm, idx_v, rows_v, *sems):
        gsems, osems = sems[:ring], sems[ring:]
        wid = lax.axis_index("s") * NC + lax.axis_index("c")
        base = wid * (nch * ch)
        pltpu.sync_copy(idx_hbm.at[wid], idx_v)

        def gstart(j, b):
            pltpu.async_copy(table_hbm.at[idx_v.at[j]], rows_v.at[b], gsems[b])

        def gwait(j, b):
            pltpu.make_async_copy(
                table_hbm.at[idx_v.at[j]], rows_v.at[b], gsems[b]
            ).wait()

        def ostart(j, b):
            pltpu.async_copy(
                rows_v.at[b],
                out_hbm.at[pl.ds(base + j * ch, ch), pl.ds(0, D)],
                osems[b],
            )

        def owait(j, b):
            pltpu.make_async_copy(
                rows_v.at[b],
                out_hbm.at[pl.ds(base + j * ch, ch), pl.ds(0, D)],
                osems[b],
            ).wait()

        for b in range(NBUF):
            gstart(b, b)

        def body(g, carry):
            for b in range(ring):
                j = g * ring + b
                gwait(j, b)
                ostart(j, b)
                bn = (b + NBUF) % ring
                jn = j + NBUF
                if b < NBUF:
                    @pl.when(g >= 1)
                    def _():
                        owait(jn - ring, bn)
                        gstart(jn, bn)

                    @pl.when(g < 1)
                    def _():
                        gstart(jn, bn)
                else:

                    @pl.when(g < ngrp - 1)
                    def _():
                        owait(jn - ring, bn)
                        gstart(jn, bn)
            return carry

        lax.fori_loop(0, ngrp, body, 0)

        for b in range(ring):
            owait(nch - ring + b, b)

    return k(table, idx3)


def kernel(variate_ids, variate_embed_weight):
    b, h = variate_ids.shape
    n = b * h
    idx3 = variate_ids.reshape(NW, n // (NW * CH), CH).astype(jnp.int32)
    out = _sc_gather(variate_embed_weight, idx3)
    return out[:, :D].reshape(b, h, D)
